# Initial kernel scaffold; baseline (speedup 1.0000x reference)
import jax
import jax.numpy as jnp
from jax.experimental import pallas as pl


def kernel(X, W, Xi):
    raise NotImplementedError("write your pallas kernel here")


# speedup vs baseline: 13.1168x; 13.1168x over previous
import jax
import jax.numpy as jnp
from jax.experimental import pallas as pl
from jax.experimental.pallas import tpu as pltpu

HEIGHT = 10
D = 2048
N_NODES = 2046
NP = 2048
BM = 256
OUTW = 128


def _fused_kernel(x_ref, w_ref, xi_ref, y_ref):
    f = jax.lax.dot_general(
        x_ref[...], w_ref[...], (((1,), (0,)), ((), ())),
        preferred_element_type=jnp.float32)
    fb = f.astype(jnp.bfloat16)
    s = jax.lax.dot_general(
        fb, xi_ref[...], (((1,), (0,)), ((), ())),
        preferred_element_type=jnp.float32)
    lane = jax.lax.broadcasted_iota(jnp.int32, (BM, NP), 1)
    y_ref[...] = jnp.zeros((BM, OUTW), jnp.int32)
    cur = jnp.zeros((BM, 1), jnp.int32)
    for h in range(HEIGHT):
        s1 = jnp.sum(jnp.where(lane == cur, s, 0.0), axis=1, keepdims=True)
        s2 = jnp.sum(jnp.where(lane == cur + 1, s, 0.0), axis=1, keepdims=True)
        b = (s1 >= s2).astype(jnp.int32)
        nxt = cur + 2 - b
        y_ref[:, h + 1:h + 2] = nxt
        cur = 2 * nxt


def kernel(X, W, Xi):
    batch = X.shape[0]
    xb = X.astype(jnp.bfloat16)
    wb = W.astype(jnp.bfloat16)
    xib = jnp.pad(Xi.astype(jnp.bfloat16), ((0, 0), (0, NP - N_NODES)))

    y = pl.pallas_call(
        _fused_kernel,
        grid=(batch // BM,),
        in_specs=[
            pl.BlockSpec((BM, D), lambda i: (i, 0)),
            pl.BlockSpec((D, D), lambda i: (0, 0)),
            pl.BlockSpec((D, NP), lambda i: (0, 0)),
        ],
        out_specs=pl.BlockSpec((BM, OUTW), lambda i: (i, 0)),
        out_shape=jax.ShapeDtypeStruct((batch, OUTW), jnp.int32),
    )(xb, wb, xib)

    return y[:, :HEIGHT + 1]


# speedup vs baseline: 18.4331x; 1.4053x over previous
import jax
import jax.numpy as jnp
from jax.experimental import pallas as pl
from jax.experimental.pallas import tpu as pltpu

HEIGHT = 10
D = 2048
N_NODES = 2046
NP = 2048
BM = 256
OUTW = 128


def _fused_kernel(x_ref, w_ref, xi_ref, y_ref):
    f = jax.lax.dot_general(
        x_ref[...].astype(jnp.bfloat16), w_ref[...],
        (((1,), (0,)), ((), ())),
        preferred_element_type=jnp.float32)
    fb = f.astype(jnp.bfloat16)
    s = jax.lax.dot_general(
        fb, xi_ref[...], (((1,), (0,)), ((), ())),
        preferred_element_type=jnp.float32)
    g = s - jnp.concatenate([s[:, 1:], s[:, :1]], axis=1)
    y_ref[...] = jnp.zeros((BM, OUTW), jnp.int32)
    cur = jnp.zeros((BM, 1), jnp.int32)
    for h in range(HEIGHT):
        w_h = 2 << h
        off = w_h - 2
        gw = g[:, off:off + w_h]
        lane = jax.lax.broadcasted_iota(jnp.int32, (BM, w_h), 1)
        d = jnp.sum(jnp.where(lane == cur - off, gw, 0.0), axis=1,
                    keepdims=True)
        b = (d >= 0).astype(jnp.int32)
        nxt = cur + 2 - b
        y_ref[:, h + 1:h + 2] = nxt
        cur = 2 * nxt


def kernel(X, W, Xi):
    batch = X.shape[0]
    wb = W.astype(jnp.bfloat16)
    xib = jnp.pad(Xi.astype(jnp.bfloat16), ((0, 0), (0, NP - N_NODES)))

    y = pl.pallas_call(
        _fused_kernel,
        grid=(batch // BM,),
        in_specs=[
            pl.BlockSpec((BM, D), lambda i: (i, 0)),
            pl.BlockSpec((D, D), lambda i: (0, 0)),
            pl.BlockSpec((D, NP), lambda i: (0, 0)),
        ],
        out_specs=pl.BlockSpec((BM, OUTW), lambda i: (i, 0)),
        out_shape=jax.ShapeDtypeStruct((batch, OUTW), jnp.int32),
    )(X, wb, xib)

    return y[:, :HEIGHT + 1]
